# baseline (speedup 1.0000x reference)
import functools
import jax
import jax.numpy as jnp
from jax import lax
from jax.experimental import pallas as pl
from jax.experimental.pallas import tpu as pltpu
from jax.experimental.pallas import tpu_sc as plsc

_D = 768
_N = 200
_TH = 0.5
_NEG = -1e30
_NPAD = 224
_HALF = 112


def _leaky(x):
    return jnp.where(x > 0, x, 0.2 * x)


def _sc_label_body(rr_hbm, labs_hbm, table_hbm, out_hbm,
                   rr_full, rr_half, idx_v, w_half, rows_v, acc_v, sem):
    h = lax.axis_index("c")
    b = lax.axis_index("s")

    pltpu.sync_copy(rr_hbm.at[pl.ds(b * _NPAD, _NPAD)], rr_full)
    tot = jnp.zeros((16,), jnp.float32)
    mx = jnp.zeros((16,), jnp.float32)
    for j in range(_NPAD // 16):
        v = rr_full[pl.ds(j * 16, 16)]
        rz = jnp.where(v < _TH, 0.0, v)
        tot = tot + rz
        mx = jnp.maximum(mx, rz)
    acc_v[pl.ds(0, 16)] = tot
    acc_v[pl.ds(16, 16)] = mx
    total = jnp.zeros((16,), jnp.float32)
    mxs = jnp.zeros((16,), jnp.float32)
    for l in range(16):
        total = total + plsc.load_gather(acc_v, [jnp.full((16,), l, jnp.int32)])
        mxs = jnp.maximum(mxs, plsc.load_gather(acc_v, [jnp.full((16,), 16 + l, jnp.int32)]))
    zero = mxs == 0.0
    inv = 1.0 / (jnp.where(zero, 1.0, total) + 1e-6)

    pltpu.sync_copy(rr_hbm.at[pl.ds(b * _NPAD + h * _HALF, _HALF)], rr_half)
    first = (h == 0) & zero
    for j in range(_HALF // 16):
        v = rr_half[pl.ds(j * 16, 16)]
        wv = jnp.where(v < _TH, 0.0, v) * inv
        if j == 0:
            lane0 = lax.iota(jnp.int32, 16) == 0
            wv = jnp.where(lane0 & first, inv, wv)
        w_half[pl.ds(j * 16, 16)] = wv

    pltpu.sync_copy(labs_hbm.at[pl.ds(b * _NPAD + h * _HALF, _HALF)], idx_v)
    pltpu.async_copy(table_hbm.at[idx_v], rows_v, sem).wait()

    for dh in range(2):
        base = dh * 384
        init = tuple(jnp.zeros((16,), jnp.float32) for _ in range(24))

        def jbody(j, accs, base=base):
            ws = plsc.load_gather(w_half, [jnp.full((16,), j, jnp.int32)])
            return tuple(a + ws * rows_v[j, pl.ds(base + k * 16, 16)]
                         for k, a in enumerate(accs))

        accs = lax.fori_loop(0, _HALF, jbody, init)
        for k in range(24):
            acc_v[pl.ds(base + k * 16, 16)] = accs[k]

    pltpu.sync_copy(acc_v, out_hbm.at[pl.ds((h * 16 + b) * _D, _D)])


def _sc_label_partials(rr_pad, labs_pad, label_table):
    mesh = plsc.VectorSubcoreMesh(core_axis_name="c", subcore_axis_name="s")
    f = functools.partial(
        pl.kernel,
        mesh=mesh,
        compiler_params=pltpu.CompilerParams(needs_layout_passes=False),
        out_type=jax.ShapeDtypeStruct((2 * 16 * _D,), jnp.float32),
        scratch_types=[
            pltpu.VMEM((_NPAD,), jnp.float32),
            pltpu.VMEM((_HALF,), jnp.float32),
            pltpu.VMEM((_HALF,), jnp.int32),
            pltpu.VMEM((_HALF,), jnp.float32),
            pltpu.VMEM((_HALF, _D), jnp.float32),
            pltpu.VMEM((_D,), jnp.float32),
            pltpu.SemaphoreType.DMA,
        ],
    )(_sc_label_body)
    return f(rr_pad, labs_pad, label_table)


def _gat_query_row(q, X, Wm, a_pair, phantom, valid_b):
    wa1 = jnp.sum(Wm * a_pair[0:1, :], axis=1)
    wa2 = jnp.sum(Wm * a_pair[1:2, :], axis=1)
    f0 = jnp.sum(q * wa1[None, :], axis=1, keepdims=True)
    gq = jnp.sum(q * wa2[None, :], axis=1, keepdims=True)
    g = jnp.sum(X * wa2[None, None, :], axis=2)
    e_q = _leaky(f0 + gq)
    e_n = _leaky(f0 + g)
    e_ph = _leaky(f0)
    e_n_m = jnp.where(valid_b, e_n, _NEG)
    m = jnp.maximum(jnp.max(e_n_m, axis=1, keepdims=True), e_q)
    m = jnp.maximum(m, jnp.where(phantom > 0, e_ph, _NEG))
    p_q = jnp.exp(e_q - m)
    p_n = jnp.where(valid_b, jnp.exp(e_n - m), 0.0)
    s = p_q + jnp.sum(p_n, axis=1, keepdims=True) + phantom * jnp.exp(e_ph - m)
    hagg = (p_q * q + jnp.sum(p_n[:, :, None] * X, axis=1)) / s
    return 0.5 * q + 0.5 * jnp.dot(hagg, Wm, preferred_element_type=jnp.float32)


def _tc_body(qt_ref, qi_ref, vis_ref, txt_ref, rr_ref,
             Wt_ref, at_ref, Wi_ref, ai_ref, Wo12_ref, bo_ref, out_ref):
    rr = rr_ref[...]
    B = rr.shape[0]
    valid_b = rr > _TH
    nvalid = jnp.sum(valid_b.astype(jnp.float32), axis=1, keepdims=True)
    is_last = lax.broadcasted_iota(jnp.int32, (B, 1), 0) == (B - 1)
    phantom = jnp.where(is_last, 0.0, _N - nvalid)

    ht0 = _gat_query_row(qt_ref[...], vis_ref[...], Wt_ref[...], at_ref[...],
                         jnp.zeros((B, 1), jnp.float32), valid_b)
    hi0 = _gat_query_row(qi_ref[...], txt_ref[...], Wi_ref[...], ai_ref[...],
                         phantom, valid_b)

    fused = jnp.concatenate([ht0, hi0], axis=1)
    out_ref[...] = jnp.dot(fused, Wo12_ref[...], preferred_element_type=jnp.float32) + bo_ref[...]


def _combine_body(p12_ref, part_ref, Wo3_ref, out_ref):
    la = part_ref[0, :, :] + part_ref[1, :, :]
    out_ref[...] = p12_ref[...] + jnp.dot(la, Wo3_ref[...],
                                          preferred_element_type=jnp.float32)


def kernel(mean_pooling_vec, merge_text_vec, retrieved_visual_feature_embedding_cls,
           retrieved_textual_feature_embedding, retrieved_label_list, RRCP,
           W_text, a_text, W_img, a_img, label_table, W_out, b_out):
    B = mean_pooling_vec.shape[0]
    vis = retrieved_visual_feature_embedding_cls[:, :_N, 0, :]
    txt = retrieved_textual_feature_embedding[:, :_N, 0, :]
    rr = RRCP[:, :_N]
    pad = jnp.zeros((B, _NPAD - _N), jnp.float32)
    rr_pad = jnp.concatenate([rr, pad], axis=1)
    labs_pad = jnp.concatenate([retrieved_label_list[:, :_N],
                                pad.astype(jnp.int32)], axis=1)

    partials = _sc_label_partials(rr_pad.reshape(-1), labs_pad.reshape(-1),
                                  label_table).reshape(2, 16, _D)

    p12 = pl.pallas_call(
        _tc_body,
        out_shape=jax.ShapeDtypeStruct((B, 2), jnp.float32),
    )(mean_pooling_vec, merge_text_vec, vis, txt, rr,
      W_text, a_text.reshape(2, _D), W_img, a_img.reshape(2, _D),
      W_out[:2 * _D], b_out.reshape(1, 2))

    return pl.pallas_call(
        _combine_body,
        out_shape=jax.ShapeDtypeStruct((B, 2), jnp.float32),
    )(p12, partials, W_out[2 * _D:])


# speedup vs baseline: 2.0739x; 2.0739x over previous
import jax
import jax.numpy as jnp
from jax import lax
from jax.experimental import pallas as pl
from jax.experimental.pallas import tpu as pltpu

_D = 768
_N = 200
_NL = 1000
_TH = 0.5
_NEG = -1e30
_CH = 40
_NS = _N // _CH


def _leaky(x):
    return jnp.where(x > 0, x, 0.2 * x)


def _wa(Wm, a_pair):
    wa1 = jnp.sum(Wm * a_pair[0:1, :], axis=1)
    wa2 = jnp.sum(Wm * a_pair[1:2, :], axis=1)
    return wa1, wa2


def _chunk_update(X, e_c, valid_c, m_ref, s_ref, acc_ref):
    m_old = m_ref[:, 0:1]
    e_m = jnp.where(valid_c, e_c, _NEG)
    m_new = jnp.maximum(m_old, jnp.max(e_m, axis=1, keepdims=True))
    scale = jnp.exp(m_old - m_new)
    p_c = jnp.where(valid_c, jnp.exp(e_c - m_new), 0.0)
    m_ref[:, 0:1] = m_new
    s_ref[:, 0:1] = s_ref[:, 0:1] * scale + jnp.sum(p_c, axis=1, keepdims=True)
    acc_ref[...] = acc_ref[...] * scale + jnp.sum(p_c[:, :, None] * X, axis=1)


def _body(qt_ref, qi_ref, vis_ref, txt_ref, rr_ref, lab_ref,
          Wt_ref, at_ref, Wi_ref, ai_ref, table_ref, Wo_ref, bo_ref,
          out_ref,
          mt_ref, st_ref, mi_ref, si_ref, nv_ref, tot_ref, zmx_ref, l0_ref,
          acct_ref, acci_ref, cnt_ref):
    j = pl.program_id(0)
    B = qt_ref.shape[0]
    qt = qt_ref[...]
    qi = qi_ref[...]
    _, wa2t = _wa(Wt_ref[...], at_ref[...])
    _, wa2i = _wa(Wi_ref[...], ai_ref[...])

    @pl.when(j == 0)
    def _init():
        wa1t = jnp.sum(Wt_ref[...] * at_ref[0:1, :], axis=1)
        wa1i = jnp.sum(Wi_ref[...] * ai_ref[0:1, :], axis=1)
        f0t = jnp.sum(qt * wa1t[None, :], axis=1, keepdims=True)
        f0i = jnp.sum(qi * wa1i[None, :], axis=1, keepdims=True)
        gqt = jnp.sum(qt * wa2t[None, :], axis=1, keepdims=True)
        gqi = jnp.sum(qi * wa2i[None, :], axis=1, keepdims=True)
        mt_ref[:, 0:1] = _leaky(f0t + gqt)
        mi_ref[:, 0:1] = _leaky(f0i + gqi)
        st_ref[:, 0:1] = jnp.ones((B, 1), jnp.float32)
        si_ref[:, 0:1] = jnp.ones((B, 1), jnp.float32)
        acct_ref[...] = qt
        acci_ref[...] = qi
        nv_ref[:, 0:1] = jnp.zeros((B, 1), jnp.float32)
        tot_ref[:, 0:1] = jnp.zeros((B, 1), jnp.float32)
        zmx_ref[:, 0:1] = jnp.zeros((B, 1), jnp.float32)
        cnt_ref[...] = jnp.zeros((B, _NL), jnp.float32)
        l0_ref[:, 0:1] = lab_ref[0][:, 0:1]

    rr_c = rr_ref[0]
    labs_c = lab_ref[0]
    valid_c = rr_c > _TH
    rrz_c = jnp.where(rr_c < _TH, 0.0, rr_c)
    nv_ref[:, 0:1] += jnp.sum(valid_c.astype(jnp.float32), axis=1, keepdims=True)
    tot_ref[:, 0:1] += jnp.sum(rrz_c, axis=1, keepdims=True)
    zmx_ref[:, 0:1] = jnp.maximum(zmx_ref[:, 0:1],
                                  jnp.max(rrz_c, axis=1, keepdims=True))

    vis = vis_ref[...]
    txt = txt_ref[...]
    wa1t = jnp.sum(Wt_ref[...] * at_ref[0:1, :], axis=1)
    wa1i = jnp.sum(Wi_ref[...] * ai_ref[0:1, :], axis=1)
    f0t = jnp.sum(qt * wa1t[None, :], axis=1, keepdims=True)
    f0i = jnp.sum(qi * wa1i[None, :], axis=1, keepdims=True)
    g_t = jnp.sum(vis * wa2t[None, None, :], axis=2)
    g_i = jnp.sum(txt * wa2i[None, None, :], axis=2)
    _chunk_update(vis, _leaky(f0t + g_t), valid_c, mt_ref, st_ref, acct_ref)
    _chunk_update(txt, _leaky(f0i + g_i), valid_c, mi_ref, si_ref, acci_ref)

    iota_c = lax.broadcasted_iota(jnp.int32, (1, 1, _NL), 2)
    cnt = cnt_ref[...]
    for s0 in range(0, _CH, 8):
        oh = labs_c[:, s0:s0 + 8, None] == iota_c
        cnt = cnt + jnp.sum(jnp.where(oh, rrz_c[:, s0:s0 + 8, None], 0.0), axis=1)
    cnt_ref[...] = cnt

    @pl.when(j == _NS - 1)
    def _final():
        nv = nv_ref[:, 0:1]
        is_last = lax.broadcasted_iota(jnp.int32, (B, 1), 0) == (B - 1)
        phantom = jnp.where(is_last, 0.0, _N - nv)
        e_ph = _leaky(f0i)
        m_old = mi_ref[:, 0:1]
        m_f = jnp.maximum(m_old, jnp.where(phantom > 0, e_ph, _NEG))
        scale = jnp.exp(m_old - m_f)
        s_i = si_ref[:, 0:1] * scale + phantom * jnp.exp(e_ph - m_f)
        acc_i = acci_ref[...] * scale
        ht0 = 0.5 * qt + 0.5 * jnp.dot(acct_ref[...] / st_ref[:, 0:1],
                                       Wt_ref[...], preferred_element_type=jnp.float32)
        hi0 = 0.5 * qi + 0.5 * jnp.dot(acc_i / s_i,
                                       Wi_ref[...], preferred_element_type=jnp.float32)
        zero = zmx_ref[:, 0:1] == 0.0
        inv = 1.0 / (jnp.where(zero, 1.0, tot_ref[:, 0:1]) + 1e-6)
        oh0 = (l0_ref[:, 0:1] == lax.broadcasted_iota(jnp.int32, (B, _NL), 1)
               ).astype(jnp.float32)
        counts = jnp.where(zero, oh0, cnt_ref[...]) * inv
        label_agg = jnp.dot(counts, table_ref[...], preferred_element_type=jnp.float32)
        fused = jnp.concatenate([ht0, hi0, label_agg], axis=1)
        out_ref[...] = jnp.dot(fused, Wo_ref[...],
                               preferred_element_type=jnp.float32) + bo_ref[...]


def kernel(mean_pooling_vec, merge_text_vec, retrieved_visual_feature_embedding_cls,
           retrieved_textual_feature_embedding, retrieved_label_list, RRCP,
           W_text, a_text, W_img, a_img, label_table, W_out, b_out):
    B = mean_pooling_vec.shape[0]
    vis = retrieved_visual_feature_embedding_cls[:, :_N, 0, :]
    txt = retrieved_textual_feature_embedding[:, :_N, 0, :]
    rrc = RRCP[:, :_N].reshape(B, _NS, _CH).swapaxes(0, 1)
    labc = retrieved_label_list[:, :_N].reshape(B, _NS, _CH).swapaxes(0, 1)

    grid = (_NS,)
    cm = lambda j: (0, 0)
    return pl.pallas_call(
        _body,
        grid=grid,
        in_specs=[
            pl.BlockSpec((B, _D), cm),
            pl.BlockSpec((B, _D), cm),
            pl.BlockSpec((B, _CH, _D), lambda j: (0, j, 0)),
            pl.BlockSpec((B, _CH, _D), lambda j: (0, j, 0)),
            pl.BlockSpec((1, B, _CH), lambda j: (j, 0, 0)),
            pl.BlockSpec((1, B, _CH), lambda j: (j, 0, 0)),
            pl.BlockSpec((_D, _D), cm),
            pl.BlockSpec((2, _D), cm),
            pl.BlockSpec((_D, _D), cm),
            pl.BlockSpec((2, _D), cm),
            pl.BlockSpec((_NL, _D), cm),
            pl.BlockSpec((3 * _D, 2), cm),
            pl.BlockSpec((1, 2), cm),
        ],
        out_specs=pl.BlockSpec((B, 2), cm),
        out_shape=jax.ShapeDtypeStruct((B, 2), jnp.float32),
        scratch_shapes=[
            pltpu.VMEM((B, 128), jnp.float32),
            pltpu.VMEM((B, 128), jnp.float32),
            pltpu.VMEM((B, 128), jnp.float32),
            pltpu.VMEM((B, 128), jnp.float32),
            pltpu.VMEM((B, 128), jnp.float32),
            pltpu.VMEM((B, 128), jnp.float32),
            pltpu.VMEM((B, 128), jnp.float32),
            pltpu.VMEM((B, 128), jnp.int32),
            pltpu.VMEM((B, _D), jnp.float32),
            pltpu.VMEM((B, _D), jnp.float32),
            pltpu.VMEM((B, _NL), jnp.float32),
        ],
    )(mean_pooling_vec, merge_text_vec, vis, txt, rrc, labc,
      W_text, a_text.reshape(2, _D), W_img, a_img.reshape(2, _D),
      label_table, W_out, b_out.reshape(1, 2))
